# Initial kernel scaffold; baseline (speedup 1.0000x reference)
import jax
import jax.numpy as jnp
from jax.experimental import pallas as pl


def kernel(detections):
    raise NotImplementedError("write your pallas kernel here")


# speedup vs baseline: 66.8916x; 66.8916x over previous
import jax
import jax.numpy as jnp
from jax import lax
from jax.experimental import pallas as pl
from jax.experimental.pallas import tpu as pltpu

N = 5120
NR = 5000
C = 128
NC = N // C
OUTW = 512
TOPK = 400
TH = 0.65
F32 = jnp.float32
I32 = jnp.int32


def _dsm(off):
    return pl.ds(pl.multiple_of(off, C), C)


def _tcol(v):
    one = jnp.ones((1, 1), F32)
    return lax.dot_general(v, one, (((0,), (0,)), ((), ())),
                           preferred_element_type=F32)


def _nms_body(dataT_ref, rows_ref, scol_ref, out_ref,
              sortedT_ref, srows_ref,
              x1c_ref, y1c_ref, x2c_ref, y2c_ref, ac_ref,
              keptc_ref, keptr_ref, posr_ref):
    srow = dataT_ref[4:5, :]
    scol = scol_ref[...]
    irow = lax.broadcasted_iota(I32, (1, N), 1)
    icol = lax.broadcasted_iota(I32, (N, 1), 0)

    def _rrow(c, acc):
        sj = scol_ref[_dsm(c * C), :]
        ij = lax.broadcasted_iota(I32, (C, 1), 0) + c * C
        before = (sj > srow) | ((sj == srow) & (ij < irow))
        return acc + jnp.sum(before.astype(F32), axis=0, keepdims=True)

    rank_r = lax.fori_loop(0, NC, _rrow, jnp.zeros((1, N), F32))

    def _rcol(c, acc):
        sj = dataT_ref[4:5, _dsm(c * C)]
        ij = lax.broadcasted_iota(I32, (1, C), 1) + c * C
        before = (sj > scol) | ((sj == scol) & (ij < icol))
        return acc + jnp.sum(before.astype(F32), axis=1, keepdims=True)

    rank_c = lax.fori_loop(0, NC, _rcol, jnp.zeros((N, 1), F32))

    dT = dataT_ref[...]
    rows = rows_ref[...]

    def _perm(c, _):
        d = _dsm(c * C)
        tr = (lax.broadcasted_iota(I32, (1, C), 1) + c * C).astype(F32)
        oh = (rank_c == tr).astype(F32)
        sortedT_ref[:, d] = jnp.dot(dT, oh, preferred_element_type=F32)
        tc_ = (lax.broadcasted_iota(I32, (C, 1), 0) + c * C).astype(F32)
        ohT = (tc_ == rank_r).astype(F32)
        rc = jnp.dot(ohT, rows, preferred_element_type=F32)
        srows_ref[d, :] = rc
        x1 = rc[:, 0:1]
        y1 = rc[:, 1:2]
        x2 = rc[:, 2:3]
        y2 = rc[:, 3:4]
        x1c_ref[d, :] = x1
        y1c_ref[d, :] = y1
        x2c_ref[d, :] = x2
        y2c_ref[d, :] = y2
        ac_ref[d, :] = (x2 - x1) * (y2 - y1)
        return 0

    lax.fori_loop(0, NC, _perm, 0)

    x1r = sortedT_ref[0:1, :]
    y1r = sortedT_ref[1:2, :]
    x2r = sortedT_ref[2:3, :]
    y2r = sortedT_ref[3:4, :]
    ar = (x2r - x1r) * (y2r - y1r)
    realr = (irow < NR).astype(F32)

    def _scond(st):
        return st[1]

    def _sbody(st):
        kept, _ = st
        keptc_ref[...] = _tcol(kept)

        def _chunk(c, sup):
            d = _dsm(c * C)
            jx1 = x1c_ref[d, :]
            jy1 = y1c_ref[d, :]
            jx2 = x2c_ref[d, :]
            jy2 = y2c_ref[d, :]
            ja = ac_ref[d, :]
            jk = keptc_ref[d, :]
            jpos = lax.broadcasted_iota(I32, (C, 1), 0) + c * C
            w = jnp.maximum(jnp.minimum(jx2, x2r) - jnp.maximum(jx1, x1r), 0.0)
            h = jnp.maximum(jnp.minimum(jy2, y2r) - jnp.maximum(jy1, y1r), 0.0)
            inter = w * h
            iou = inter / (ja + ar - inter)
            hit = (iou > TH) & (jpos < irow) & (jk > 0.5)
            return jnp.maximum(sup, jnp.max(hit.astype(F32), axis=0,
                                            keepdims=True))

        sup = lax.fori_loop(0, NC, _chunk, jnp.zeros((1, N), F32))
        new = (1.0 - sup) * realr
        return (new, jnp.any(new != kept))

    kept, _ = lax.while_loop(_scond, _sbody, (realr, jnp.bool_(True)))

    keptr_ref[...] = kept
    tri = (lax.broadcasted_iota(I32, (C, C), 0)
           < lax.broadcasted_iota(I32, (C, C), 1)).astype(F32)

    def _pos(c, run):
        d = _dsm(c * C)
        kch = keptr_ref[:, d]
        posr_ref[:, d] = jnp.dot(kch, tri, preferred_element_type=F32) + run
        return run + jnp.sum(kch)

    count = lax.fori_loop(0, NC, _pos, jnp.float32(0.0))

    tgtc = lax.broadcasted_iota(I32, (OUTW, 1), 0).astype(F32)
    hsel = ((posr_ref[...] == tgtc)
            & (keptr_ref[...] > 0.5)).astype(F32)
    outv = jnp.dot(hsel, srows_ref[...], preferred_element_type=F32)
    out_ref[...] = jnp.where(tgtc < count, outv, -1.0)


def kernel(detections):
    det = detections.astype(F32)
    rows = jnp.zeros((N, 8), F32)
    rows = rows.at[:, 4].set(-1.0)
    rows = rows.at[:NR, :5].set(det)
    dataT = rows.T
    scol = rows[:, 4:5]
    out = pl.pallas_call(
        _nms_body,
        out_shape=jax.ShapeDtypeStruct((OUTW, 8), F32),
        scratch_shapes=[
            pltpu.VMEM((8, N), F32),
            pltpu.VMEM((N, 8), F32),
            pltpu.VMEM((N, 1), F32),
            pltpu.VMEM((N, 1), F32),
            pltpu.VMEM((N, 1), F32),
            pltpu.VMEM((N, 1), F32),
            pltpu.VMEM((N, 1), F32),
            pltpu.VMEM((N, 1), F32),
            pltpu.VMEM((1, N), F32),
            pltpu.VMEM((1, N), F32),
        ],
    )(dataT, rows, scol)
    return out[:TOPK, :5]


# speedup vs baseline: 221.4122x; 3.3100x over previous
import jax
import jax.numpy as jnp
from jax import lax
from jax.experimental import pallas as pl
from jax.experimental.pallas import tpu as pltpu

N = 5120
NR = 5000
C = 128
NC = N // C
TIERS = 4
BT = NC // TIERS
OUTW = 512
TOPK = 400
TH = 0.65
F32 = jnp.float32
I32 = jnp.int32


def _dsm(off):
    return pl.ds(pl.multiple_of(off, C), C)


def _tcol(v):
    one = jnp.ones((1, 1), F32)
    return lax.dot_general(v, one, (((0,), (0,)), ((), ())),
                           preferred_element_type=F32)


def _nms_body(dataT_ref, rows_ref, scol_ref, out_ref,
              sortedT_ref, srows_ref,
              x1c_ref, y1c_ref, x2c_ref, y2c_ref, ac_ref,
              keptc_ref, keptr_ref, posr_ref):
    srow = dataT_ref[4:5, :]
    irow = lax.broadcasted_iota(I32, (1, N), 1)

    def _rrow(c, acc):
        sj = scol_ref[_dsm(c * C), :]
        ij = lax.broadcasted_iota(I32, (C, 1), 0) + c * C
        before = (sj > srow) | ((sj == srow) & (ij < irow))
        return acc + jnp.sum(before.astype(F32), axis=0, keepdims=True)

    rank_r = lax.fori_loop(0, NC, _rrow, jnp.zeros((1, N), F32))

    rows = rows_ref[...]

    def _perm(c, _):
        d = _dsm(c * C)
        tc_ = (lax.broadcasted_iota(I32, (C, 1), 0) + c * C).astype(F32)
        ohT = (tc_ == rank_r).astype(F32)
        rc = jnp.dot(ohT, rows, preferred_element_type=F32)
        srows_ref[d, :] = rc
        x1 = rc[:, 0:1]
        y1 = rc[:, 1:2]
        x2 = rc[:, 2:3]
        y2 = rc[:, 3:4]
        x1c_ref[d, :] = x1
        y1c_ref[d, :] = y1
        x2c_ref[d, :] = x2
        y2c_ref[d, :] = y2
        ac_ref[d, :] = (x2 - x1) * (y2 - y1)
        return 0

    lax.fori_loop(0, NC, _perm, 0)

    eye8 = (lax.broadcasted_iota(I32, (8, 8), 0)
            == lax.broadcasted_iota(I32, (8, 8), 1)).astype(F32)
    sortedT_ref[...] = lax.dot_general(
        eye8, srows_ref[...], (((1,), (1,)), ((), ())),
        preferred_element_type=F32)

    for t in range(TIERS):
        H = (t + 1) * (N // TIERS)

        def _blk(bl, _, t=t, H=H):
            b = t * BT + bl
            d = _dsm(b * C)
            bx1 = sortedT_ref[0:1, d]
            by1 = sortedT_ref[1:2, d]
            bx2 = sortedT_ref[2:3, d]
            by2 = sortedT_ref[3:4, d]
            ba = (bx2 - bx1) * (by2 - by1)
            ipos = lax.broadcasted_iota(I32, (1, C), 1) + b * C

            jx1 = x1c_ref[0:H, :]
            jy1 = y1c_ref[0:H, :]
            jx2 = x2c_ref[0:H, :]
            jy2 = y2c_ref[0:H, :]
            ja = ac_ref[0:H, :]
            jk = keptc_ref[0:H, :]
            jpos = lax.broadcasted_iota(I32, (H, 1), 0)
            w = jnp.maximum(jnp.minimum(jx2, bx2) - jnp.maximum(jx1, bx1),
                            0.0)
            h = jnp.maximum(jnp.minimum(jy2, by2) - jnp.maximum(jy1, by1),
                            0.0)
            inter = w * h
            union = ja + ba - inter
            hit = ((inter > TH * union) & (union >= 0.0)
                   & (jpos < b * C) & (jk > 0.5))
            ext = jnp.max(hit.astype(F32), axis=0, keepdims=True)

            lx1 = x1c_ref[d, :]
            ly1 = y1c_ref[d, :]
            lx2 = x2c_ref[d, :]
            ly2 = y2c_ref[d, :]
            la = ac_ref[d, :]
            jloc = lax.broadcasted_iota(I32, (C, C), 0)
            iloc = lax.broadcasted_iota(I32, (C, C), 1)
            w2 = jnp.maximum(jnp.minimum(lx2, bx2) - jnp.maximum(lx1, bx1),
                             0.0)
            h2 = jnp.maximum(jnp.minimum(ly2, by2) - jnp.maximum(ly1, by1),
                             0.0)
            inter2 = w2 * h2
            union2 = la + ba - inter2
            lhit = ((inter2 > TH * union2) & (union2 >= 0.0)
                    & (jloc < iloc)).astype(F32)

            realb = (ipos < NR).astype(F32)
            kb0 = (ext < 0.5).astype(F32) * realb

            def _lcond(st):
                return st[1]

            def _lbody(st):
                kb, _ = st
                cnt = jnp.dot(kb, lhit, preferred_element_type=F32)
                new = ((ext + cnt) < 0.5).astype(F32) * realb
                return (new, jnp.any(new != kb))

            kb, _ = lax.while_loop(_lcond, _lbody, (kb0, jnp.bool_(True)))
            keptr_ref[:, d] = kb
            keptc_ref[d, :] = _tcol(kb)
            return 0

        lax.fori_loop(0, BT, _blk, 0)

    tri = (lax.broadcasted_iota(I32, (C, C), 0)
           < lax.broadcasted_iota(I32, (C, C), 1)).astype(F32)

    def _pos(c, run):
        d = _dsm(c * C)
        kch = keptr_ref[:, d]
        posr_ref[:, d] = jnp.dot(kch, tri, preferred_element_type=F32) + run
        return run + jnp.sum(kch)

    count = lax.fori_loop(0, NC, _pos, jnp.float32(0.0))

    tgtc = lax.broadcasted_iota(I32, (OUTW, 1), 0).astype(F32)
    hsel = ((posr_ref[...] == tgtc)
            & (keptr_ref[...] > 0.5)).astype(F32)
    outv = jnp.dot(hsel, srows_ref[...], preferred_element_type=F32)
    out_ref[...] = jnp.where(tgtc < count, outv, -1.0)


def kernel(detections):
    det = detections.astype(F32)
    rows = jnp.zeros((N, 8), F32)
    rows = rows.at[:, 4].set(-1.0)
    rows = rows.at[:NR, :5].set(det)
    dataT = rows.T
    scol = rows[:, 4:5]
    out = pl.pallas_call(
        _nms_body,
        out_shape=jax.ShapeDtypeStruct((OUTW, 8), F32),
        scratch_shapes=[
            pltpu.VMEM((8, N), F32),
            pltpu.VMEM((N, 8), F32),
            pltpu.VMEM((N, 1), F32),
            pltpu.VMEM((N, 1), F32),
            pltpu.VMEM((N, 1), F32),
            pltpu.VMEM((N, 1), F32),
            pltpu.VMEM((N, 1), F32),
            pltpu.VMEM((N, 1), F32),
            pltpu.VMEM((1, N), F32),
            pltpu.VMEM((1, N), F32),
        ],
    )(dataT, rows, scol)
    return out[:TOPK, :5]
